# baseline (speedup 1.0000x reference)
import functools

import jax
import jax.numpy as jnp
from jax import lax
from jax.experimental import pallas as pl
from jax.experimental.pallas import tpu as pltpu
from jax.experimental.pallas import tpu_sc as plsc

N = 10000
E = 320000
D = 128

NC = 2
NS = 16
DH = D // NC
CH = 125
NCH = (E // NS) // CH
NB = 2
NP = 10240
RPT = NP // NS
ZB = 128
DEGW = 16
WIN = (0, 16, 32, 48, 64, 80, 96, CH - 16)


def _sc_segment_sum(with_deg: bool):
    mesh = plsc.VectorSubcoreMesh(core_axis_name="c", subcore_axis_name="s")

    out_type = [jax.ShapeDtypeStruct((NC, NP, DH), jnp.float32)]
    if with_deg:
        out_type.append(jax.ShapeDtypeStruct((NC, NP, DEGW), jnp.float32))

    scratch = [
        pltpu.VMEM((NCH, CH), jnp.int32),
        pltpu.VMEM((NCH, CH), jnp.int32),
        pltpu.VMEM((NB, CH, DH), jnp.float32),
        pltpu.VMEM((ZB, DH), jnp.float32),
        pltpu.VMEM((CH, DEGW), jnp.float32),
        pltpu.VMEM((ZB, DEGW), jnp.float32),
        pltpu.VMEM_SHARED((NP, DH), jnp.float32),
        pltpu.VMEM_SHARED((NP, DEGW), jnp.float32),
    ] + [pltpu.SemaphoreType.DMA] * (2 * NB)

    def body(table, src2, dst2, *rest):
        if with_deg:
            out, degout = rest[0], rest[1]
            scr = rest[2:]
        else:
            out = rest[0]
            scr = rest[1:]
        gidx, dstv, rows, zrows, ones, zeros, acc, dacc = scr[:8]
        gsem = scr[8:8 + NB]
        ssem = scr[8 + NB:8 + 2 * NB]

        cid = lax.axis_index("c")
        sid = lax.axis_index("s")
        deg_lo = cid * (NCH // 2)
        deg_hi = deg_lo + (NCH // 2)

        pltpu.sync_copy(src2.at[sid], gidx)
        pltpu.sync_copy(dst2.at[sid], dstv)

        zero16 = jnp.zeros((16,), jnp.float32)
        one16 = jnp.ones((16,), jnp.float32)

        def fill_ones(i, _):
            ones[i, :] = one16
            return 0

        def fill_zeros(i, _):
            zeros[i, :] = zero16
            for j in range(DH // 16):
                zrows[i, pl.ds(16 * j, 16)] = zero16
            return 0

        lax.fori_loop(0, CH, fill_ones, 0)
        lax.fori_loop(0, ZB, fill_zeros, 0)

        def fill_gidx(i, _):
            tail = gidx[i, pl.ds(CH - 16, 16)]
            for o in WIN[:-1]:
                v = gidx[i, pl.ds(o, 16)]
                gidx[i, pl.ds(o, 16)] = v + v + cid
            gidx[i, pl.ds(CH - 16, 16)] = tail + tail + cid
            return 0

        lax.fori_loop(0, NCH, fill_gidx, 0)

        for k in range(RPT // ZB):
            pltpu.sync_copy(zrows, acc.at[pl.ds(sid * RPT + k * ZB, ZB)])
            if with_deg:
                pltpu.sync_copy(zeros, dacc.at[pl.ds(sid * RPT + k * ZB, ZB)])
        plsc.subcore_barrier()

        def gather(j, b):
            pltpu.async_copy(table.at[gidx.at[j]], rows.at[b], gsem[b])

        def wait_gather(j, b):
            pltpu.make_async_copy(table.at[gidx.at[j]], rows.at[b], gsem[b]).wait()

        def scatter(j, b):
            pltpu.async_copy(rows.at[b], acc.at[dstv.at[j]], ssem[b], add=True)
            if with_deg:
                @pl.when(jnp.logical_and(j >= deg_lo, j < deg_hi))
                def _():
                    pltpu.sync_copy(ones, dacc.at[dstv.at[j]], add=True)

        def wait_scatter(j, b):
            pltpu.make_async_copy(rows.at[b], acc.at[dstv.at[j]], ssem[b]).wait()

        for b in range(NB):
            gather(b, b)

        def step(g, _):
            j0 = NB * g
            for b in range(NB):
                j = j0 + b
                wait_gather(j, b)
                scatter(j, b)

                @pl.when(j + NB < NCH)
                def _():
                    wait_scatter(j, b)
                    gather(jnp.minimum(j + NB, NCH - 1), b)
            return 0

        lax.fori_loop(0, NCH // NB, step, 0)

        for b in range(NB):
            wait_scatter(NCH - NB + b, b)
        plsc.subcore_barrier()

        pltpu.sync_copy(acc.at[pl.ds(sid * RPT, RPT)],
                        out.at[cid, pl.ds(sid * RPT, RPT)])
        if with_deg:
            pltpu.sync_copy(dacc.at[pl.ds(sid * RPT, RPT)],
                            degout.at[cid, pl.ds(sid * RPT, RPT)])


    return pl.kernel(
        body,
        out_type=out_type,
        mesh=mesh,
        scratch_types=scratch,
        compiler_params=pltpu.CompilerParams(use_tc_tiling_on_sc=False),
    )


def _tc_self():
    R = 1000

    def body(x_ref, ws_ref, b_ref, o_ref):
        o_ref[...] = (
            jnp.dot(x_ref[...], ws_ref[...], precision=lax.Precision.HIGHEST,
                    preferred_element_type=jnp.float32)
            + b_ref[...]
        )

    return pl.pallas_call(
        body,
        grid=(N // R,),
        in_specs=[
            pl.BlockSpec((R, D), lambda i: (i, 0)),
            pl.BlockSpec((D, D), lambda i: (0, 0)),
            pl.BlockSpec((1, D), lambda i: (0, 0)),
        ],
        out_specs=pl.BlockSpec((R, D), lambda i: (i, 0)),
        out_shape=jax.ShapeDtypeStruct((N, D), jnp.float32),
    )


def _tc_combine(apply_relu: bool):
    R = 1000

    def body(xw_ref, p_ref, d_ref, wn_ref, o_ref):
        p = p_ref[...]
        summed = jnp.concatenate([p[0], p[1]], axis=-1)
        d = d_ref[0, :, 0:1] + d_ref[1, :, 0:1]
        mean = summed * (1.0 / jnp.maximum(d, 1.0))
        y = xw_ref[...] + jnp.dot(
            mean, wn_ref[...], precision=lax.Precision.HIGHEST,
            preferred_element_type=jnp.float32)
        if apply_relu:
            y = jnp.maximum(y, 0.0)
        o_ref[...] = y

    return pl.pallas_call(
        body,
        grid=(N // R,),
        in_specs=[
            pl.BlockSpec((R, D), lambda i: (i, 0)),
            pl.BlockSpec((NC, R, DH), lambda i: (0, i, 0)),
            pl.BlockSpec((NC, R, DEGW), lambda i: (0, i, 0)),
            pl.BlockSpec((D, D), lambda i: (0, 0)),
        ],
        out_specs=pl.BlockSpec((R, D), lambda i: (i, 0)),
        out_shape=jax.ShapeDtypeStruct((N, D), jnp.float32),
    )


_seg_sum_deg = _sc_segment_sum(with_deg=True)
_seg_sum = _sc_segment_sum(with_deg=False)
_self_mm = _tc_self()
_combine_relu = _tc_combine(apply_relu=True)
_combine_lin = _tc_combine(apply_relu=False)


def kernel(x, edge_index, W_self1, W_neigh1, b1, W_self2, W_neigh2, b2):
    src2 = edge_index[0].astype(jnp.int32).reshape(NS, NCH, CH)
    dst2 = edge_index[1].astype(jnp.int32).reshape(NS, NCH, CH)
    b1r = b1.reshape(1, D)
    b2r = b2.reshape(1, D)

    psum1, deg = _seg_sum_deg(x.reshape(2 * N, DH), src2, dst2)
    xw1 = _self_mm(x, W_self1, b1r)
    h = _combine_relu(xw1, psum1, deg, W_neigh1)
    (psum2,) = _seg_sum(h.reshape(2 * N, DH), src2, dst2)
    hw2 = _self_mm(h, W_self2, b2r)
    out = _combine_lin(hw2, psum2, deg, W_neigh2)
    return out


# speedup vs baseline: 1.0206x; 1.0206x over previous
import functools

import jax
import jax.numpy as jnp
from jax import lax
from jax.experimental import pallas as pl
from jax.experimental.pallas import tpu as pltpu
from jax.experimental.pallas import tpu_sc as plsc

N = 10000
E = 320000
D = 128

NC = 2
NS = 16
DH = D // NC
CH = 125
NCH = (E // NS) // CH
NB = 2
NP = 10240
RPT = NP // NS
ZB = 128
DEGW = 16
WIN = (0, 16, 32, 48, 64, 80, 96, CH - 16)


def _sc_segment_sum(with_deg: bool):
    mesh = plsc.VectorSubcoreMesh(core_axis_name="c", subcore_axis_name="s")

    out_type = [jax.ShapeDtypeStruct((NC, NP, DH), jnp.float32)]
    if with_deg:
        out_type.append(jax.ShapeDtypeStruct((NC, NP, DEGW), jnp.float32))

    scratch = [
        pltpu.VMEM((NCH, CH), jnp.int32),
        pltpu.VMEM((NCH, CH), jnp.int32),
        pltpu.VMEM((NB, CH, DH), jnp.float32),
        pltpu.VMEM((ZB, DH), jnp.float32),
        pltpu.VMEM((CH, DEGW), jnp.float32),
        pltpu.VMEM((ZB, DEGW), jnp.float32),
        pltpu.VMEM_SHARED((NP, DH), jnp.float32),
        pltpu.VMEM_SHARED((NP, DEGW), jnp.float32),
    ] + [pltpu.SemaphoreType.DMA] * (2 * NB)

    def body(table, src2, dst2, *rest):
        if with_deg:
            out, degout = rest[0], rest[1]
            scr = rest[2:]
        else:
            out = rest[0]
            scr = rest[1:]
        gidx, dstv, rows, zrows, ones, zeros, acc, dacc = scr[:8]
        gsem = scr[8:8 + NB]
        ssem = scr[8 + NB:8 + 2 * NB]

        cid = lax.axis_index("c")
        sid = lax.axis_index("s")
        deg_lo = cid * (NCH // 2)
        deg_hi = deg_lo + (NCH // 2)

        pltpu.sync_copy(src2.at[sid], gidx)
        pltpu.sync_copy(dst2.at[sid], dstv)

        zero16 = jnp.zeros((16,), jnp.float32)
        one16 = jnp.ones((16,), jnp.float32)

        def fill_ones(i, _):
            ones[i, :] = one16
            return 0

        def fill_zeros(i, _):
            zeros[i, :] = zero16
            for j in range(DH // 16):
                zrows[i, pl.ds(16 * j, 16)] = zero16
            return 0

        lax.fori_loop(0, CH, fill_ones, 0)
        lax.fori_loop(0, ZB, fill_zeros, 0)

        def fill_gidx(i, _):
            tail = gidx[i, pl.ds(CH - 16, 16)]
            for o in WIN[:-1]:
                v = gidx[i, pl.ds(o, 16)]
                gidx[i, pl.ds(o, 16)] = v + v + cid
            gidx[i, pl.ds(CH - 16, 16)] = tail + tail + cid
            return 0

        lax.fori_loop(0, NCH, fill_gidx, 0)

        for k in range(RPT // ZB):
            pltpu.sync_copy(zrows, acc.at[pl.ds(sid * RPT + k * ZB, ZB)])
            if with_deg:
                pltpu.sync_copy(zeros, dacc.at[pl.ds(sid * RPT + k * ZB, ZB)])
        plsc.subcore_barrier()

        def gather(j, b):
            pltpu.async_copy(table.at[gidx.at[j]], rows.at[b], gsem[b])

        def wait_gather(j, b):
            pltpu.make_async_copy(table.at[gidx.at[j]], rows.at[b], gsem[b]).wait()

        def scatter(j, b):
            pltpu.async_copy(rows.at[b], acc.at[dstv.at[j]], ssem[b], add=True)
            if with_deg:
                @pl.when(jnp.logical_and(j >= deg_lo, j < deg_hi))
                def _():
                    pltpu.sync_copy(ones, dacc.at[dstv.at[j]], add=True)

        def wait_scatter(j, b):
            pltpu.make_async_copy(rows.at[b], acc.at[dstv.at[j]], ssem[b]).wait()

        for b in range(NB):
            gather(b, b)

        def step(g, _):
            j0 = NB * g
            for b in range(NB):
                j = j0 + b
                wait_gather(j, b)
                scatter(j, b)

                @pl.when(j + NB < NCH)
                def _():
                    wait_scatter(j, b)
                    gather(jnp.minimum(j + NB, NCH - 1), b)
            return 0

        lax.fori_loop(0, NCH // NB, step, 0)

        for b in range(NB):
            wait_scatter(NCH - NB + b, b)
        plsc.subcore_barrier()

        pltpu.sync_copy(acc.at[pl.ds(sid * RPT, RPT)],
                        out.at[cid, pl.ds(sid * RPT, RPT)])
        if with_deg:
            pltpu.sync_copy(dacc.at[pl.ds(sid * RPT, RPT)],
                            degout.at[cid, pl.ds(sid * RPT, RPT)])


    return pl.kernel(
        body,
        out_type=out_type,
        mesh=mesh,
        scratch_types=scratch,
        compiler_params=pltpu.CompilerParams(use_tc_tiling_on_sc=False),
    )


def _tc_self():
    R = 2000

    def body(x_ref, ws_ref, b_ref, o_ref):
        o_ref[...] = (
            jnp.dot(x_ref[...], ws_ref[...], precision=lax.Precision.HIGHEST,
                    preferred_element_type=jnp.float32)
            + b_ref[...]
        )

    return pl.pallas_call(
        body,
        grid=(N // R,),
        in_specs=[
            pl.BlockSpec((R, D), lambda i: (i, 0)),
            pl.BlockSpec((D, D), lambda i: (0, 0)),
            pl.BlockSpec((1, D), lambda i: (0, 0)),
        ],
        out_specs=pl.BlockSpec((R, D), lambda i: (i, 0)),
        out_shape=jax.ShapeDtypeStruct((N, D), jnp.float32),
    )


def _tc_combine(apply_relu: bool):
    R = 2000

    def body(xw_ref, p_ref, d_ref, wn_ref, o_ref):
        p = p_ref[...]
        summed = jnp.concatenate([p[0], p[1]], axis=-1)
        d = d_ref[0, :, 0:1] + d_ref[1, :, 0:1]
        mean = summed * (1.0 / jnp.maximum(d, 1.0))
        y = xw_ref[...] + jnp.dot(
            mean, wn_ref[...], precision=lax.Precision.HIGHEST,
            preferred_element_type=jnp.float32)
        if apply_relu:
            y = jnp.maximum(y, 0.0)
        o_ref[...] = y

    return pl.pallas_call(
        body,
        grid=(N // R,),
        in_specs=[
            pl.BlockSpec((R, D), lambda i: (i, 0)),
            pl.BlockSpec((NC, R, DH), lambda i: (0, i, 0)),
            pl.BlockSpec((NC, R, DEGW), lambda i: (0, i, 0)),
            pl.BlockSpec((D, D), lambda i: (0, 0)),
        ],
        out_specs=pl.BlockSpec((R, D), lambda i: (i, 0)),
        out_shape=jax.ShapeDtypeStruct((N, D), jnp.float32),
    )


_seg_sum_deg = _sc_segment_sum(with_deg=True)
_seg_sum = _sc_segment_sum(with_deg=False)
_self_mm = _tc_self()
_combine_relu = _tc_combine(apply_relu=True)
_combine_lin = _tc_combine(apply_relu=False)


def kernel(x, edge_index, W_self1, W_neigh1, b1, W_self2, W_neigh2, b2):
    src2 = edge_index[0].astype(jnp.int32).reshape(NS, NCH, CH)
    dst2 = edge_index[1].astype(jnp.int32).reshape(NS, NCH, CH)
    b1r = b1.reshape(1, D)
    b2r = b2.reshape(1, D)

    psum1, deg = _seg_sum_deg(x.reshape(2 * N, DH), src2, dst2)
    xw1 = _self_mm(x, W_self1, b1r)
    h = _combine_relu(xw1, psum1, deg, W_neigh1)
    (psum2,) = _seg_sum(h.reshape(2 * N, DH), src2, dst2)
    hw2 = _self_mm(h, W_self2, b2r)
    out = _combine_lin(hw2, psum2, deg, W_neigh2)
    return out
